# baseline (speedup 1.0000x reference)
import functools
import jax
import jax.numpy as jnp
import numpy as np
from jax.experimental import pallas as pl
from jax.experimental.pallas import tpu as pltpu

N = 16384
K = 20
NUM_GRAPHS = 16
CHUNK = 2048


_KNN_R = 256
_KNN_CT = 512
_KNN_NT = N // _KNN_CT


def _knn_body(lo_ref, hi_ref, x_ref, xT_ref, bT_ref, bC_ref, wc_ref, wq_ref,
              bc_ref, out_ref, c_out_ref, q_out_ref, ds_ref):
    R, CT = _KNN_R, _KNN_CT
    r = pl.program_id(0)
    lo_t = lo_ref[r]
    hi_t = hi_ref[r]

    xr = x_ref[pl.ds(r * R, R), :]
    sq_r = jnp.sum(xr * xr, axis=1, keepdims=True)
    b_r = bT_ref[...]
    row_gid = r * R + jax.lax.broadcasted_iota(jnp.int32, (R, 1), 0)

    c_out_ref[...] = jnp.dot(xr, wc_ref[...],
                             preferred_element_type=jnp.float32) + bc_ref[...]
    q_out_ref[...] = jnp.dot(xr, wq_ref[...],
                             preferred_element_type=jnp.float32)

    inf = jnp.float32(jnp.inf)
    big_i = jnp.int32(2**30)
    lane_t = jax.lax.broadcasted_iota(jnp.int32, (R, _KNN_NT), 1)
    col_base = jax.lax.broadcasted_iota(jnp.int32, (R, CT), 1)

    def fill_tile(t, MA):
        M, A = MA
        c0 = t * CT
        xc = xT_ref[:, pl.ds(c0, CT)]
        dot = jnp.dot(xr, xc, preferred_element_type=jnp.float32)
        sq_c = jnp.sum(xc * xc, axis=0, keepdims=True)
        d = sq_r + sq_c - 2.0 * dot
        b_c = bC_ref[:, pl.ds(c0, CT)]
        col = c0 + col_base
        d = jnp.where((b_r != b_c) | (col == row_gid), inf, d)
        ds_ref[:, pl.ds(c0, CT)] = d
        mt = jnp.min(d, axis=1, keepdims=True)
        at = jnp.min(jnp.where(d == mt, col, big_i), axis=1, keepdims=True)
        M = jnp.where(lane_t == t, mt, M)
        A = jnp.where(lane_t == t, at, A)
        return M, A

    M, A = jax.lax.fori_loop(
        lo_t, hi_t, fill_tile,
        (jnp.full((R, _KNN_NT), inf, jnp.float32),
         jnp.full((R, _KNN_NT), big_i, jnp.int32)))

    for kk in range(K):
        m = jnp.min(M, axis=1, keepdims=True)
        c = jnp.min(jnp.where(M == m, A, big_i), axis=1, keepdims=True)
        out_ref[:, kk : kk + 1] = jnp.minimum(c, jnp.int32(N - 1))
        if kk == K - 1:
            break

        def tile_scan(t, MA):
            M, A = MA
            c0 = t * CT
            dt = ds_ref[:, pl.ds(c0, CT)]
            col = c0 + col_base
            dm = jnp.where((dt > m) | ((dt == m) & (col > c)), dt, inf)
            vt = jnp.min(dm, axis=1, keepdims=True)
            at = jnp.min(jnp.where(dm == vt, col, big_i),
                         axis=1, keepdims=True)
            M = jnp.where(lane_t == t, vt, M)
            A = jnp.where(lane_t == t, at, A)
            return M, A

        M, A = jax.lax.fori_loop(lo_t, hi_t, tile_scan, (M, A))


def _knn_prep(x, batch, W, b):
    n, d = x.shape
    F = W.shape[1]
    Wc = W[:d] - W[d:]
    Wq = W[d:]
    dp = 8 if d < 8 else d
    if d != dp:
        x = jnp.pad(x, ((0, 0), (0, dp - d)))
        Wc = jnp.pad(Wc, ((0, dp - d), (0, 0)))
        Wq = jnp.pad(Wq, ((0, dp - d), (0, 0)))
    xT = x.T
    bT = batch.reshape(n, 1)
    bC = batch.reshape(1, n)

    nb = n // _KNN_R
    seg_start = jnp.searchsorted(batch, jnp.arange(NUM_GRAPHS), side="left")
    seg_end = jnp.searchsorted(batch, jnp.arange(NUM_GRAPHS), side="right")
    b_lo = batch[:: _KNN_R]
    b_hi = batch[_KNN_R - 1 :: _KNN_R]
    lo_t = (seg_start[b_lo] // _KNN_CT).astype(jnp.int32)
    hi_t = ((seg_end[b_hi] + _KNN_CT - 1) // _KNN_CT).astype(jnp.int32)

    nbr, C, Q = pl.pallas_call(
        _knn_body,
        grid_spec=pltpu.PrefetchScalarGridSpec(
            num_scalar_prefetch=2,
            grid=(nb,),
            in_specs=[
                pl.BlockSpec((n, dp), lambda r, lo, hi: (0, 0)),
                pl.BlockSpec((dp, n), lambda r, lo, hi: (0, 0)),
                pl.BlockSpec((_KNN_R, 1), lambda r, lo, hi: (r, 0)),
                pl.BlockSpec((1, n), lambda r, lo, hi: (0, 0)),
                pl.BlockSpec((dp, F), lambda r, lo, hi: (0, 0)),
                pl.BlockSpec((dp, F), lambda r, lo, hi: (0, 0)),
                pl.BlockSpec((1, F), lambda r, lo, hi: (0, 0)),
            ],
            out_specs=[
                pl.BlockSpec((_KNN_R, 32), lambda r, lo, hi: (r, 0)),
                pl.BlockSpec((_KNN_R, F), lambda r, lo, hi: (r, 0)),
                pl.BlockSpec((_KNN_R, F), lambda r, lo, hi: (r, 0)),
            ],
            scratch_shapes=[pltpu.VMEM((_KNN_R, n), jnp.float32)],
        ),
        compiler_params=pltpu.CompilerParams(
            dimension_semantics=("parallel",)),
        out_shape=[
            jax.ShapeDtypeStruct((n, 32), jnp.int32),
            jax.ShapeDtypeStruct((n, F), jnp.float32),
            jax.ShapeDtypeStruct((n, F), jnp.float32),
        ],
    )(lo_t, hi_t, x, xT, bT, bC, Wc, Wq, b.reshape(1, F))
    return nbr[:, :K].T, C, Q


def _ec_body(c_ref, g_ref, w2_ref, b2_ref, w3_ref, b3_ref, o_ref):
    c = c_ref[...]
    acc = None
    for k in range(K):
        h = jnp.maximum(c + g_ref[k], 0.0)
        h = jnp.maximum(
            jnp.dot(h, w2_ref[...], preferred_element_type=jnp.float32)
            + b2_ref[...], 0.0)
        h = jnp.maximum(
            jnp.dot(h, w3_ref[...], preferred_element_type=jnp.float32)
            + b3_ref[...], 0.0)
        acc = h if acc is None else jnp.maximum(acc, h)
    o_ref[...] = acc


def _edge_conv(C, G, W2, b2, W3, b3, R=512):
    Din = C.shape[1]
    Dmid = W2.shape[1]
    Dout = W3.shape[1]
    b2 = b2.reshape(1, Dmid)
    b3 = b3.reshape(1, Dout)
    return pl.pallas_call(
        _ec_body,
        grid=(N // R,),
        in_specs=[
            pl.BlockSpec((R, Din), lambda r: (r, 0)),
            pl.BlockSpec((K, R, Din), lambda r: (0, r, 0)),
            pl.BlockSpec((Din, Dmid), lambda r: (0, 0)),
            pl.BlockSpec((1, Dmid), lambda r: (0, 0)),
            pl.BlockSpec((Dmid, Dout), lambda r: (0, 0)),
            pl.BlockSpec((1, Dout), lambda r: (0, 0)),
        ],
        out_specs=pl.BlockSpec((R, Dout), lambda r: (r, 0)),
        out_shape=jax.ShapeDtypeStruct((N, Dout), jnp.float32),
        compiler_params=pltpu.CompilerParams(
            dimension_semantics=("parallel",)),
    )(C, G, W2, b2, W3, b3)


def kernel(pos, batch, W1a, b1a, W1b, b1b, W1c, b1c, W2a, b2a, W2b, b2b,
           W2c, b2c, W0, b0, Wl1, bl1, Wl2, bl2, Wl3, bl3):
    nbr1, C1, Q1 = _knn_prep(pos, batch, W1a, b1a)
    G1 = Q1[nbr1]
    x1 = _edge_conv(C1, G1, W1b, b1b, W1c, b1c)

    nbr2, C2, Q2 = _knn_prep(x1, batch, W2a, b2a)
    G2 = Q2[nbr2]
    x2 = _edge_conv(C2, G2, W2b, b2b, W2c, b2c)

    y = jax.nn.relu(x2 @ W0 + b0)
    y = jax.ops.segment_max(y, batch, num_segments=NUM_GRAPHS)
    y = jax.nn.relu(y @ Wl1 + bl1)
    y = jax.nn.relu(y @ Wl2 + bl2)
    y = y @ Wl3 + bl3
    return jax.nn.log_softmax(y, axis=-1)


# speedup vs baseline: 1.5147x; 1.5147x over previous
import functools
import jax
import jax.numpy as jnp
import numpy as np
from jax.experimental import pallas as pl
from jax.experimental.pallas import tpu as pltpu
from jax.experimental.pallas import tpu_sc as plsc

N = 16384
K = 20
NUM_GRAPHS = 16
CHUNK = 2048


_KNN_R = 256
_KNN_CT = 512
_KNN_NT = N // _KNN_CT


def _knn_body(lo_ref, hi_ref, x_ref, xT_ref, bT_ref, bC_ref, wc_ref, wq_ref,
              bc_ref, out_ref, c_out_ref, q_out_ref, ds_ref):
    R, CT = _KNN_R, _KNN_CT
    r = pl.program_id(0)
    lo_t = lo_ref[r]
    hi_t = hi_ref[r]

    xr = x_ref[pl.ds(r * R, R), :]
    sq_r = jnp.sum(xr * xr, axis=1, keepdims=True)
    b_r = bT_ref[...]
    row_gid = r * R + jax.lax.broadcasted_iota(jnp.int32, (R, 1), 0)

    c_out_ref[...] = jnp.dot(xr, wc_ref[...],
                             preferred_element_type=jnp.float32) + bc_ref[...]
    q_out_ref[...] = jnp.dot(xr, wq_ref[...],
                             preferred_element_type=jnp.float32)

    inf = jnp.float32(jnp.inf)
    big_i = jnp.int32(2**30)
    lane_t = jax.lax.broadcasted_iota(jnp.int32, (R, _KNN_NT), 1)
    col_base = jax.lax.broadcasted_iota(jnp.int32, (R, CT), 1)

    def fill_tile(t, MA):
        M, A = MA
        c0 = t * CT
        xc = xT_ref[:, pl.ds(c0, CT)]
        dot = jnp.dot(xr, xc, preferred_element_type=jnp.float32)
        sq_c = jnp.sum(xc * xc, axis=0, keepdims=True)
        d = sq_r + sq_c - 2.0 * dot
        b_c = bC_ref[:, pl.ds(c0, CT)]
        col = c0 + col_base
        d = jnp.maximum(d, 0.0)
        d = jnp.where((b_r != b_c) | (col == row_gid), inf, d)
        ds_ref[:, pl.ds(c0, CT)] = d
        mt = jnp.min(d, axis=1, keepdims=True)
        at = c0 + jnp.min(jnp.where(d == mt, col_base, big_i),
                          axis=1, keepdims=True)
        M = jnp.where(lane_t == t, mt, M)
        A = jnp.where(lane_t == t, at, A)
        return M, A

    M, A = jax.lax.fori_loop(
        lo_t, hi_t, fill_tile,
        (jnp.full((R, _KNN_NT), inf, jnp.float32),
         jnp.full((R, _KNN_NT), big_i, jnp.int32)))

    for kk in range(K):
        m = jnp.min(M, axis=1, keepdims=True)
        c = jnp.min(jnp.where(M == m, A, big_i), axis=1, keepdims=True)
        out_ref[:, kk : kk + 1] = jnp.minimum(c, jnp.int32(N - 1))
        if kk == K - 1:
            break

        m_plus = jax.lax.bitcast_convert_type(
            jax.lax.bitcast_convert_type(m, jnp.int32) + 1, jnp.float32)

        def tile_scan(t, MA):
            M, A = MA
            c0 = t * CT
            dt = ds_ref[:, pl.ds(c0, CT)]
            cc = c - c0
            thr = jnp.where(col_base > cc, m, m_plus)
            dm = jnp.where(dt >= thr, dt, inf)
            vt = jnp.min(dm, axis=1, keepdims=True)
            at = c0 + jnp.min(jnp.where(dm == vt, col_base, big_i),
                              axis=1, keepdims=True)
            M = jnp.where(lane_t == t, vt, M)
            A = jnp.where(lane_t == t, at, A)
            return M, A

        M, A = jax.lax.fori_loop(lo_t, hi_t, tile_scan, (M, A))


def _knn_prep(x, batch, W, b):
    n, d = x.shape
    F = W.shape[1]
    Wc = W[:d] - W[d:]
    Wq = W[d:]
    dp = 8 if d < 8 else d
    if d != dp:
        x = jnp.pad(x, ((0, 0), (0, dp - d)))
        Wc = jnp.pad(Wc, ((0, dp - d), (0, 0)))
        Wq = jnp.pad(Wq, ((0, dp - d), (0, 0)))
    xT = x.T
    bT = batch.reshape(n, 1)
    bC = batch.reshape(1, n)

    nb = n // _KNN_R
    seg_start = jnp.searchsorted(batch, jnp.arange(NUM_GRAPHS), side="left")
    seg_end = jnp.searchsorted(batch, jnp.arange(NUM_GRAPHS), side="right")
    b_lo = batch[:: _KNN_R]
    b_hi = batch[_KNN_R - 1 :: _KNN_R]
    lo_t = (seg_start[b_lo] // _KNN_CT).astype(jnp.int32)
    hi_t = ((seg_end[b_hi] + _KNN_CT - 1) // _KNN_CT).astype(jnp.int32)

    nbr, C, Q = pl.pallas_call(
        _knn_body,
        grid_spec=pltpu.PrefetchScalarGridSpec(
            num_scalar_prefetch=2,
            grid=(nb,),
            in_specs=[
                pl.BlockSpec((n, dp), lambda r, lo, hi: (0, 0)),
                pl.BlockSpec((dp, n), lambda r, lo, hi: (0, 0)),
                pl.BlockSpec((_KNN_R, 1), lambda r, lo, hi: (r, 0)),
                pl.BlockSpec((1, n), lambda r, lo, hi: (0, 0)),
                pl.BlockSpec((dp, F), lambda r, lo, hi: (0, 0)),
                pl.BlockSpec((dp, F), lambda r, lo, hi: (0, 0)),
                pl.BlockSpec((1, F), lambda r, lo, hi: (0, 0)),
            ],
            out_specs=[
                pl.BlockSpec((_KNN_R, 32), lambda r, lo, hi: (r, 0)),
                pl.BlockSpec((_KNN_R, F), lambda r, lo, hi: (r, 0)),
                pl.BlockSpec((_KNN_R, F), lambda r, lo, hi: (r, 0)),
            ],
            scratch_shapes=[pltpu.VMEM((_KNN_R, n), jnp.float32)],
        ),
        compiler_params=pltpu.CompilerParams(
            dimension_semantics=("parallel",)),
        out_shape=[
            jax.ShapeDtypeStruct((n, 32), jnp.int32),
            jax.ShapeDtypeStruct((n, F), jnp.float32),
            jax.ShapeDtypeStruct((n, F), jnp.float32),
        ],
    )(lo_t, hi_t, x, xT, bT, bC, Wc, Wq, b.reshape(1, F))
    return nbr[:, :K].T, C, Q


def _sc_gather(Q, nbr):
    K_, n = nbr.shape
    F = Q.shape[1]
    B = K_ * n
    info = plsc.get_sparse_core_info()
    NC, NS = info.num_cores, info.num_subcores
    NW = NC * NS
    b_per_w = B // NW
    CH = 256
    n_ch = b_per_w // CH
    assert b_per_w % CH == 0 and B % NW == 0 and b_per_w % 8 == 0
    mesh = plsc.VectorSubcoreMesh(core_axis_name="c", subcore_axis_name="s")

    @functools.partial(
        pl.kernel, mesh=mesh,
        out_type=jax.ShapeDtypeStruct((B, F), jnp.float32),
        scratch_types=[
            pltpu.VMEM((b_per_w,), jnp.int32),
            pltpu.VMEM((CH, F), jnp.float32),
            pltpu.VMEM((CH, F), jnp.float32),
            pltpu.SemaphoreType.DMA,
            pltpu.SemaphoreType.DMA,
        ],
    )
    def gk(q_hbm, idx_hbm, out_hbm, idx_v, rows0, rows1, sem0, sem1):
        wid = jax.lax.axis_index("s") * NC + jax.lax.axis_index("c")
        base = wid * b_per_w
        pltpu.sync_copy(idx_hbm.at[pl.ds(base, b_per_w)], idx_v)

        def body(g, _):
            i0 = 2 * g * CH
            i1 = (2 * g + 1) * CH
            h0 = pltpu.async_copy(
                q_hbm.at[idx_v.at[pl.ds(i0, CH)]], rows0, sem0)
            h1 = pltpu.async_copy(
                q_hbm.at[idx_v.at[pl.ds(i1, CH)]], rows1, sem1)
            h0.wait()
            pltpu.sync_copy(rows0, out_hbm.at[pl.ds(base + i0, CH)])
            h1.wait()
            pltpu.sync_copy(rows1, out_hbm.at[pl.ds(base + i1, CH)])
            return 0

        jax.lax.fori_loop(0, n_ch // 2, body, 0)

    G = gk(Q, nbr.reshape(B))
    return G.reshape(K_, n, F)


def _ec_body(c_ref, g_ref, w2_ref, b2_ref, w3_ref, b3_ref, o_ref):
    c = c_ref[...]
    din = c.shape[1]
    acc = None
    for k in range(K):
        h = jnp.maximum(c + g_ref[k][:, :din], 0.0)
        h = jnp.maximum(
            jnp.dot(h, w2_ref[...], preferred_element_type=jnp.float32)
            + b2_ref[...], 0.0)
        h = jnp.maximum(
            jnp.dot(h, w3_ref[...], preferred_element_type=jnp.float32)
            + b3_ref[...], 0.0)
        acc = h if acc is None else jnp.maximum(acc, h)
    o_ref[...] = acc


def _edge_conv(C, G, W2, b2, W3, b3, R=512):
    Din = C.shape[1]
    Fg = G.shape[2]
    Dmid = W2.shape[1]
    Dout = W3.shape[1]
    b2 = b2.reshape(1, Dmid)
    b3 = b3.reshape(1, Dout)
    return pl.pallas_call(
        _ec_body,
        grid=(N // R,),
        in_specs=[
            pl.BlockSpec((R, Din), lambda r: (r, 0)),
            pl.BlockSpec((K, R, Fg), lambda r: (0, r, 0)),
            pl.BlockSpec((Din, Dmid), lambda r: (0, 0)),
            pl.BlockSpec((1, Dmid), lambda r: (0, 0)),
            pl.BlockSpec((Dmid, Dout), lambda r: (0, 0)),
            pl.BlockSpec((1, Dout), lambda r: (0, 0)),
        ],
        out_specs=pl.BlockSpec((R, Dout), lambda r: (r, 0)),
        out_shape=jax.ShapeDtypeStruct((N, Dout), jnp.float32),
        compiler_params=pltpu.CompilerParams(
            dimension_semantics=("parallel",)),
    )(C, G, W2, b2, W3, b3)


def kernel(pos, batch, W1a, b1a, W1b, b1b, W1c, b1c, W2a, b2a, W2b, b2b,
           W2c, b2c, W0, b0, Wl1, bl1, Wl2, bl2, Wl3, bl3):
    nbr1, C1, Q1 = _knn_prep(pos, batch, W1a, b1a)
    Q1p = jnp.pad(Q1, ((0, 0), (0, 64)))
    G1 = _sc_gather(Q1p, nbr1)
    x1 = _edge_conv(C1, G1, W1b, b1b, W1c, b1c)

    nbr2, C2, Q2 = _knn_prep(x1, batch, W2a, b2a)
    G2 = _sc_gather(Q2, nbr2)
    x2 = _edge_conv(C2, G2, W2b, b2b, W2c, b2c)

    y = jax.nn.relu(x2 @ W0 + b0)
    y = jax.ops.segment_max(y, batch, num_segments=NUM_GRAPHS)
    y = jax.nn.relu(y @ Wl1 + bl1)
    y = jax.nn.relu(y @ Wl2 + bl2)
    y = y @ Wl3 + bl3
    return jax.nn.log_softmax(y, axis=-1)


# speedup vs baseline: 2.6942x; 1.7787x over previous
import functools
import jax
import jax.numpy as jnp
import numpy as np
from jax.experimental import pallas as pl
from jax.experimental.pallas import tpu as pltpu
from jax.experimental.pallas import tpu_sc as plsc

N = 16384
K = 20
NUM_GRAPHS = 16
CHUNK = 2048


_KNN_R = 256
_KNN_CT = 256
_KNN_NT = N // _KNN_CT


def _knn_body(lo_ref, hi_ref, x_ref, xT_ref, bT_ref, bC_ref, wc_ref, wq_ref,
              bc_ref, out_ref, c_out_ref, q_out_ref, ds_ref):
    R, CT = _KNN_R, _KNN_CT
    r = pl.program_id(0)
    lo_t = lo_ref[r]
    hi_t = hi_ref[r]

    xr = x_ref[pl.ds(r * R, R), :]
    b_r = bC_ref[:, pl.ds(r * R, R)]
    row_gid = r * R + jax.lax.broadcasted_iota(jnp.int32, (1, R), 1)

    c_out_ref[...] = jnp.dot(xr, wc_ref[...],
                             preferred_element_type=jnp.float32) + bc_ref[...]
    q_out_ref[...] = jnp.dot(xr, wq_ref[...],
                             preferred_element_type=jnp.float32)

    inf = jnp.float32(jnp.inf)
    big_i = jnp.int32(2**30)
    sub_t = jax.lax.broadcasted_iota(jnp.int32, (_KNN_NT, R), 0)
    col_base = jax.lax.broadcasted_iota(jnp.int32, (CT, 1), 0)
    xrT = xT_ref[:, pl.ds(r * R, R)]
    sq_r = jnp.sum(xrT * xrT, axis=0, keepdims=True)

    def fill_tile(t, MA):
        M, A = MA
        c0 = t * CT
        xc = x_ref[pl.ds(c0, CT), :]
        dot = jnp.dot(xc, xrT, preferred_element_type=jnp.float32)
        sq_c = jnp.sum(xc * xc, axis=1, keepdims=True)
        d = sq_c + sq_r - 2.0 * dot
        b_c = bT_ref[pl.ds(c0, CT), :]
        col = c0 + col_base
        d = jnp.maximum(d, 0.0)
        d = jnp.where((b_c != b_r) | (col == row_gid), inf, d)
        ds_ref[pl.ds(c0, CT), :] = d
        mt = jnp.min(d, axis=0, keepdims=True)
        at = jnp.min(jnp.where(d == mt, col, big_i),
                     axis=0, keepdims=True)
        M = jnp.where(sub_t == t, mt, M)
        A = jnp.where(sub_t == t, at, A)
        return M, A

    M, A = jax.lax.fori_loop(
        lo_t, hi_t, fill_tile,
        (jnp.full((_KNN_NT, R), inf, jnp.float32),
         jnp.full((_KNN_NT, R), big_i, jnp.int32)))

    for kk in range(K):
        m = jnp.min(M, axis=0, keepdims=True)
        c = jnp.min(jnp.where(M == m, A, big_i), axis=0, keepdims=True)
        out_ref[kk : kk + 1, :] = jnp.minimum(c, jnp.int32(N - 1))
        if kk == K - 1:
            break

        m_plus = jax.lax.bitcast_convert_type(
            jax.lax.bitcast_convert_type(m, jnp.int32) + 1, jnp.float32)

        def tile_scan(t, MA):
            M, A = MA
            c0 = t * CT
            dt = ds_ref[pl.ds(c0, CT), :]
            cc = c - c0
            thr = jnp.where(col_base > cc, m, m_plus)
            dm = jnp.where(dt >= thr, dt, inf)
            vt = jnp.min(dm, axis=0, keepdims=True)
            at = c0 + jnp.min(jnp.where(dm == vt, col_base, big_i),
                              axis=0, keepdims=True)
            M = jnp.where(sub_t == t, vt, M)
            A = jnp.where(sub_t == t, at, A)
            return M, A

        M, A = jax.lax.fori_loop(lo_t, hi_t, tile_scan, (M, A))


def _knn_prep(x, batch, W, b):
    n, d = x.shape
    F = W.shape[1]
    Wc = W[:d] - W[d:]
    Wq = W[d:]
    dp = 8 if d < 8 else d
    if d != dp:
        x = jnp.pad(x, ((0, 0), (0, dp - d)))
        Wc = jnp.pad(Wc, ((0, dp - d), (0, 0)))
        Wq = jnp.pad(Wq, ((0, dp - d), (0, 0)))
    xT = x.T
    bT = batch.reshape(n, 1)
    bC = batch.reshape(1, n)

    nb = n // _KNN_R
    seg_start = jnp.searchsorted(batch, jnp.arange(NUM_GRAPHS), side="left")
    seg_end = jnp.searchsorted(batch, jnp.arange(NUM_GRAPHS), side="right")
    b_lo = batch[:: _KNN_R]
    b_hi = batch[_KNN_R - 1 :: _KNN_R]
    lo_t = (seg_start[b_lo] // _KNN_CT).astype(jnp.int32)
    hi_t = ((seg_end[b_hi] + _KNN_CT - 1) // _KNN_CT).astype(jnp.int32)

    nbr, C, Q = pl.pallas_call(
        _knn_body,
        grid_spec=pltpu.PrefetchScalarGridSpec(
            num_scalar_prefetch=2,
            grid=(nb,),
            in_specs=[
                pl.BlockSpec((n, dp), lambda r, lo, hi: (0, 0)),
                pl.BlockSpec((dp, n), lambda r, lo, hi: (0, 0)),
                pl.BlockSpec((n, 1), lambda r, lo, hi: (0, 0)),
                pl.BlockSpec((1, n), lambda r, lo, hi: (0, 0)),
                pl.BlockSpec((dp, F), lambda r, lo, hi: (0, 0)),
                pl.BlockSpec((dp, F), lambda r, lo, hi: (0, 0)),
                pl.BlockSpec((1, F), lambda r, lo, hi: (0, 0)),
            ],
            out_specs=[
                pl.BlockSpec((32, _KNN_R), lambda r, lo, hi: (0, r)),
                pl.BlockSpec((_KNN_R, F), lambda r, lo, hi: (r, 0)),
                pl.BlockSpec((_KNN_R, F), lambda r, lo, hi: (r, 0)),
            ],
            scratch_shapes=[pltpu.VMEM((n, _KNN_R), jnp.float32)],
        ),
        compiler_params=pltpu.CompilerParams(
            dimension_semantics=("parallel",)),
        out_shape=[
            jax.ShapeDtypeStruct((32, n), jnp.int32),
            jax.ShapeDtypeStruct((n, F), jnp.float32),
            jax.ShapeDtypeStruct((n, F), jnp.float32),
        ],
    )(lo_t, hi_t, x, xT, bT, bC, Wc, Wq, b.reshape(1, F))
    return nbr[:K], C, Q


def _sc_gather(Q, nbr):
    K_, n = nbr.shape
    F = Q.shape[1]
    B = K_ * n
    info = plsc.get_sparse_core_info()
    NC, NS = info.num_cores, info.num_subcores
    NW = NC * NS
    b_per_w = B // NW
    CH = 256
    n_ch = b_per_w // CH
    assert b_per_w % CH == 0 and B % NW == 0 and b_per_w % 8 == 0
    mesh = plsc.VectorSubcoreMesh(core_axis_name="c", subcore_axis_name="s")

    @functools.partial(
        pl.kernel, mesh=mesh,
        out_type=jax.ShapeDtypeStruct((B, F), jnp.float32),
        scratch_types=[
            pltpu.VMEM((b_per_w,), jnp.int32),
            pltpu.VMEM((CH, F), jnp.float32),
            pltpu.VMEM((CH, F), jnp.float32),
            pltpu.SemaphoreType.DMA,
            pltpu.SemaphoreType.DMA,
        ],
    )
    def gk(q_hbm, idx_hbm, out_hbm, idx_v, rows0, rows1, sem0, sem1):
        wid = jax.lax.axis_index("s") * NC + jax.lax.axis_index("c")
        base = wid * b_per_w
        pltpu.sync_copy(idx_hbm.at[pl.ds(base, b_per_w)], idx_v)

        def body(g, _):
            i0 = 2 * g * CH
            i1 = (2 * g + 1) * CH
            h0 = pltpu.async_copy(
                q_hbm.at[idx_v.at[pl.ds(i0, CH)]], rows0, sem0)
            h1 = pltpu.async_copy(
                q_hbm.at[idx_v.at[pl.ds(i1, CH)]], rows1, sem1)
            h0.wait()
            pltpu.sync_copy(rows0, out_hbm.at[pl.ds(base + i0, CH)])
            h1.wait()
            pltpu.sync_copy(rows1, out_hbm.at[pl.ds(base + i1, CH)])
            return 0

        jax.lax.fori_loop(0, n_ch // 2, body, 0)

    G = gk(Q, nbr.reshape(B))
    return G.reshape(K_, n, F)


def _ec_body(c_ref, g_ref, w2_ref, b2_ref, w3_ref, b3_ref, o_ref):
    c = c_ref[...]
    din = c.shape[1]
    acc = None
    for k in range(K):
        h = jnp.maximum(c + g_ref[k][:, :din], 0.0)
        h = jnp.maximum(
            jnp.dot(h, w2_ref[...], preferred_element_type=jnp.float32)
            + b2_ref[...], 0.0)
        h = jnp.maximum(
            jnp.dot(h, w3_ref[...], preferred_element_type=jnp.float32)
            + b3_ref[...], 0.0)
        acc = h if acc is None else jnp.maximum(acc, h)
    o_ref[...] = acc


def _edge_conv(C, G, W2, b2, W3, b3, R=512):
    Din = C.shape[1]
    Fg = G.shape[2]
    Dmid = W2.shape[1]
    Dout = W3.shape[1]
    b2 = b2.reshape(1, Dmid)
    b3 = b3.reshape(1, Dout)
    return pl.pallas_call(
        _ec_body,
        grid=(N // R,),
        in_specs=[
            pl.BlockSpec((R, Din), lambda r: (r, 0)),
            pl.BlockSpec((K, R, Fg), lambda r: (0, r, 0)),
            pl.BlockSpec((Din, Dmid), lambda r: (0, 0)),
            pl.BlockSpec((1, Dmid), lambda r: (0, 0)),
            pl.BlockSpec((Dmid, Dout), lambda r: (0, 0)),
            pl.BlockSpec((1, Dout), lambda r: (0, 0)),
        ],
        out_specs=pl.BlockSpec((R, Dout), lambda r: (r, 0)),
        out_shape=jax.ShapeDtypeStruct((N, Dout), jnp.float32),
        compiler_params=pltpu.CompilerParams(
            dimension_semantics=("parallel",)),
    )(C, G, W2, b2, W3, b3)


def kernel(pos, batch, W1a, b1a, W1b, b1b, W1c, b1c, W2a, b2a, W2b, b2b,
           W2c, b2c, W0, b0, Wl1, bl1, Wl2, bl2, Wl3, bl3):
    nbr1, C1, Q1 = _knn_prep(pos, batch, W1a, b1a)
    Q1p = jnp.pad(Q1, ((0, 0), (0, 64)))
    G1 = _sc_gather(Q1p, nbr1)
    x1 = _edge_conv(C1, G1, W1b, b1b, W1c, b1c)

    nbr2, C2, Q2 = _knn_prep(x1, batch, W2a, b2a)
    G2 = _sc_gather(Q2, nbr2)
    x2 = _edge_conv(C2, G2, W2b, b2b, W2c, b2c)

    y = jax.nn.relu(x2 @ W0 + b0)
    y = jax.ops.segment_max(y, batch, num_segments=NUM_GRAPHS)
    y = jax.nn.relu(y @ Wl1 + bl1)
    y = jax.nn.relu(y @ Wl2 + bl2)
    y = y @ Wl3 + bl3
    return jax.nn.log_softmax(y, axis=-1)


# speedup vs baseline: 2.8697x; 1.0651x over previous
import functools
import jax
import jax.numpy as jnp
import numpy as np
from jax.experimental import pallas as pl
from jax.experimental.pallas import tpu as pltpu
from jax.experimental.pallas import tpu_sc as plsc

N = 16384
K = 20
NUM_GRAPHS = 16
CHUNK = 2048


_KNN_R = 256
_KNN_CT = 256
_KNN_NT = N // _KNN_CT


def _knn_body(lo_ref, hi_ref, x_ref, xT_ref, bT_ref, bC_ref, wc_ref, wq_ref,
              bc_ref, out_ref, c_out_ref, q_out_ref, ds_ref):
    R, CT = _KNN_R, _KNN_CT
    r = pl.program_id(0)
    lo_t = lo_ref[r]
    hi_t = hi_ref[r]

    xr = x_ref[pl.ds(r * R, R), :]
    b_r = bC_ref[:, pl.ds(r * R, R)]
    row_gid = r * R + jax.lax.broadcasted_iota(jnp.int32, (1, R), 1)

    c_out_ref[...] = jnp.dot(xr, wc_ref[...],
                             preferred_element_type=jnp.float32) + bc_ref[...]
    q_out_ref[...] = jnp.dot(xr, wq_ref[...],
                             preferred_element_type=jnp.float32)

    inf = jnp.float32(jnp.inf)
    big_i = jnp.int32(2**30)
    sub_t = jax.lax.broadcasted_iota(jnp.int32, (_KNN_NT, R), 0)
    col_base = jax.lax.broadcasted_iota(jnp.int32, (CT, 1), 0)
    xrT = xT_ref[:, pl.ds(r * R, R)]
    sq_r = jnp.sum(xrT * xrT, axis=0, keepdims=True)

    def fill_tile(t, MA):
        M, A = MA
        c0 = t * CT
        xc = x_ref[pl.ds(c0, CT), :]
        dot = jnp.dot(xc, xrT, preferred_element_type=jnp.float32)
        sq_c = jnp.sum(xc * xc, axis=1, keepdims=True)
        d = sq_c + sq_r - 2.0 * dot
        b_c = bT_ref[pl.ds(c0, CT), :]
        col = c0 + col_base
        d = jnp.maximum(d, 0.0)
        d = jnp.where((b_c != b_r) | (col == row_gid), inf, d)
        ds_ref[pl.ds(c0, CT), :] = d
        mt = jnp.min(d, axis=0, keepdims=True)
        at = jnp.min(jnp.where(d == mt, col, big_i),
                     axis=0, keepdims=True)
        M = jnp.where(sub_t == t, mt, M)
        A = jnp.where(sub_t == t, at, A)
        return M, A

    M, A = jax.lax.fori_loop(
        lo_t, hi_t, fill_tile,
        (jnp.full((_KNN_NT, R), inf, jnp.float32),
         jnp.full((_KNN_NT, R), big_i, jnp.int32)))

    for kk in range(K):
        m = jnp.min(M, axis=0, keepdims=True)
        c = jnp.min(jnp.where(M == m, A, big_i), axis=0, keepdims=True)
        out_ref[kk : kk + 1, :] = jnp.minimum(c, jnp.int32(N - 1))
        if kk == K - 1:
            break

        m_plus = jax.lax.bitcast_convert_type(
            jax.lax.bitcast_convert_type(m, jnp.int32) + 1, jnp.float32)

        def tile_scan(t, MA):
            M, A = MA
            c0 = t * CT
            dt = ds_ref[pl.ds(c0, CT), :]
            cc = c - c0
            thr = jnp.where(col_base > cc, m, m_plus)
            dm = jnp.where(dt >= thr, dt, inf)
            vt = jnp.min(dm, axis=0, keepdims=True)
            at = c0 + jnp.min(jnp.where(dm == vt, col_base, big_i),
                              axis=0, keepdims=True)
            M = jnp.where(sub_t == t, vt, M)
            A = jnp.where(sub_t == t, at, A)
            return M, A

        M, A = jax.lax.fori_loop(lo_t, hi_t, tile_scan, (M, A))


def _knn_prep(x, batch, W, b):
    n, d = x.shape
    F = W.shape[1]
    Wc = W[:d] - W[d:]
    Wq = W[d:]
    dp = 8 if d < 8 else d
    if d != dp:
        x = jnp.pad(x, ((0, 0), (0, dp - d)))
        Wc = jnp.pad(Wc, ((0, dp - d), (0, 0)))
        Wq = jnp.pad(Wq, ((0, dp - d), (0, 0)))
    xT = x.T
    bT = batch.reshape(n, 1)
    bC = batch.reshape(1, n)

    nb = n // _KNN_R
    seg_start = jnp.searchsorted(batch, jnp.arange(NUM_GRAPHS), side="left")
    seg_end = jnp.searchsorted(batch, jnp.arange(NUM_GRAPHS), side="right")
    b_lo = batch[:: _KNN_R]
    b_hi = batch[_KNN_R - 1 :: _KNN_R]
    lo_t = (seg_start[b_lo] // _KNN_CT).astype(jnp.int32)
    hi_t = ((seg_end[b_hi] + _KNN_CT - 1) // _KNN_CT).astype(jnp.int32)

    nbr, C, Q = pl.pallas_call(
        _knn_body,
        grid_spec=pltpu.PrefetchScalarGridSpec(
            num_scalar_prefetch=2,
            grid=(nb,),
            in_specs=[
                pl.BlockSpec((n, dp), lambda r, lo, hi: (0, 0)),
                pl.BlockSpec((dp, n), lambda r, lo, hi: (0, 0)),
                pl.BlockSpec((n, 1), lambda r, lo, hi: (0, 0)),
                pl.BlockSpec((1, n), lambda r, lo, hi: (0, 0)),
                pl.BlockSpec((dp, F), lambda r, lo, hi: (0, 0)),
                pl.BlockSpec((dp, F), lambda r, lo, hi: (0, 0)),
                pl.BlockSpec((1, F), lambda r, lo, hi: (0, 0)),
            ],
            out_specs=[
                pl.BlockSpec((32, _KNN_R), lambda r, lo, hi: (0, r)),
                pl.BlockSpec((_KNN_R, F), lambda r, lo, hi: (r, 0)),
                pl.BlockSpec((_KNN_R, F), lambda r, lo, hi: (r, 0)),
            ],
            scratch_shapes=[pltpu.VMEM((n, _KNN_R), jnp.float32)],
        ),
        compiler_params=pltpu.CompilerParams(
            dimension_semantics=("parallel",)),
        out_shape=[
            jax.ShapeDtypeStruct((32, n), jnp.int32),
            jax.ShapeDtypeStruct((n, F), jnp.float32),
            jax.ShapeDtypeStruct((n, F), jnp.float32),
        ],
    )(lo_t, hi_t, x, xT, bT, bC, Wc, Wq, b.reshape(1, F))
    return nbr[:K], C, Q


def _sc_gather(Q, nbr):
    K_, n = nbr.shape
    F = Q.shape[1]
    B = K_ * n
    info = plsc.get_sparse_core_info()
    NC, NS = info.num_cores, info.num_subcores
    NW = NC * NS
    b_per_w = B // NW
    CH = 256
    n_ch = b_per_w // CH
    assert b_per_w % CH == 0 and B % NW == 0 and b_per_w % 8 == 0
    mesh = plsc.VectorSubcoreMesh(core_axis_name="c", subcore_axis_name="s")

    @functools.partial(
        pl.kernel, mesh=mesh,
        out_type=jax.ShapeDtypeStruct((B, F), jnp.float32),
        scratch_types=[
            pltpu.VMEM((b_per_w,), jnp.int32),
            pltpu.VMEM((CH, F), jnp.float32),
            pltpu.VMEM((CH, F), jnp.float32),
            pltpu.SemaphoreType.DMA,
            pltpu.SemaphoreType.DMA,
        ],
    )
    def gk(q_hbm, idx_hbm, out_hbm, idx_v, rows0, rows1, sem0, sem1):
        wid = jax.lax.axis_index("s") * NC + jax.lax.axis_index("c")
        base = wid * b_per_w
        pltpu.sync_copy(idx_hbm.at[pl.ds(base, b_per_w)], idx_v)

        def body(g, _):
            i0 = 2 * g * CH
            i1 = (2 * g + 1) * CH
            h0 = pltpu.async_copy(
                q_hbm.at[idx_v.at[pl.ds(i0, CH)]], rows0, sem0)
            h1 = pltpu.async_copy(
                q_hbm.at[idx_v.at[pl.ds(i1, CH)]], rows1, sem1)
            h0.wait()
            pltpu.sync_copy(rows0, out_hbm.at[pl.ds(base + i0, CH)])
            h1.wait()
            pltpu.sync_copy(rows1, out_hbm.at[pl.ds(base + i1, CH)])
            return 0

        jax.lax.fori_loop(0, n_ch // 2, body, 0)

    G = gk(Q, nbr.reshape(B))
    return G.reshape(K_, n, F)


def _ec_body(c_ref, g_ref, w2_ref, b2_ref, w3_ref, b3_ref, o_ref):
    c = c_ref[...]
    din = c.shape[1]
    acc = None
    for k in range(K):
        h = jnp.maximum(c + g_ref[k][:, :din], 0.0)
        h = jnp.maximum(
            jnp.dot(h, w2_ref[...], preferred_element_type=jnp.float32)
            + b2_ref[...], 0.0)
        h = jnp.maximum(
            jnp.dot(h, w3_ref[...], preferred_element_type=jnp.float32)
            + b3_ref[...], 0.0)
        acc = h if acc is None else jnp.maximum(acc, h)
    o_ref[...] = acc


def _edge_conv(C, G, W2, b2, W3, b3, R=512):
    Din = C.shape[1]
    Fg = G.shape[2]
    Dmid = W2.shape[1]
    Dout = W3.shape[1]
    b2 = b2.reshape(1, Dmid)
    b3 = b3.reshape(1, Dout)
    return pl.pallas_call(
        _ec_body,
        grid=(N // R,),
        in_specs=[
            pl.BlockSpec((R, Din), lambda r: (r, 0)),
            pl.BlockSpec((K, R, Fg), lambda r: (0, r, 0)),
            pl.BlockSpec((Din, Dmid), lambda r: (0, 0)),
            pl.BlockSpec((1, Dmid), lambda r: (0, 0)),
            pl.BlockSpec((Dmid, Dout), lambda r: (0, 0)),
            pl.BlockSpec((1, Dout), lambda r: (0, 0)),
        ],
        out_specs=pl.BlockSpec((R, Dout), lambda r: (r, 0)),
        out_shape=jax.ShapeDtypeStruct((N, Dout), jnp.float32),
        compiler_params=pltpu.CompilerParams(
            dimension_semantics=("parallel",)),
    )(C, G, W2, b2, W3, b3)


_HD_R = 512


def _head_body(glo_ref, ghi_ref, x_ref, bT_ref, w0_ref, b0_ref, wl1_ref,
               bl1_ref, wl2_ref, bl2_ref, wl3_ref, bl3_ref, out_ref, acc_ref):
    r = pl.program_id(0)
    nb = pl.num_programs(0)
    ninf = jnp.float32(-jnp.inf)

    y = jnp.maximum(
        jnp.dot(x_ref[...], w0_ref[...],
                preferred_element_type=jnp.float32) + b0_ref[...], 0.0)

    @pl.when(r == 0)
    def _():
        acc_ref[...] = jnp.full_like(acc_ref, ninf)

    b_blk = bT_ref[...]

    def upd(g, _):
        mx = jnp.max(jnp.where(b_blk == g, y, ninf), axis=0, keepdims=True)
        acc_ref[pl.ds(g, 1), :] = jnp.maximum(acc_ref[pl.ds(g, 1), :], mx)
        return 0

    jax.lax.fori_loop(glo_ref[r], ghi_ref[r], upd, 0)

    @pl.when(r == nb - 1)
    def _():
        z = jnp.maximum(
            jnp.dot(acc_ref[...], wl1_ref[...],
                    preferred_element_type=jnp.float32) + bl1_ref[...], 0.0)
        z = jnp.maximum(
            jnp.dot(z, wl2_ref[...],
                    preferred_element_type=jnp.float32) + bl2_ref[...], 0.0)
        z = jnp.dot(z, wl3_ref[...],
                    preferred_element_type=jnp.float32) + bl3_ref[...]
        mx = jnp.max(z, axis=1, keepdims=True)
        z = z - mx
        out_ref[...] = z - jnp.log(
            jnp.sum(jnp.exp(z), axis=1, keepdims=True))


def _head(x2, batch, W0, b0, Wl1, bl1, Wl2, bl2, Wl3, bl3):
    R = _HD_R
    nb = N // R
    NC = Wl3.shape[1]
    glo = batch[::R].astype(jnp.int32)
    ghi = (batch[R - 1 :: R] + 1).astype(jnp.int32)
    bT = batch.reshape(N, 1)
    return pl.pallas_call(
        _head_body,
        grid_spec=pltpu.PrefetchScalarGridSpec(
            num_scalar_prefetch=2,
            grid=(nb,),
            in_specs=[
                pl.BlockSpec((R, 256), lambda r, glo, ghi: (r, 0)),
                pl.BlockSpec((R, 1), lambda r, glo, ghi: (r, 0)),
                pl.BlockSpec((256, 512), lambda r, glo, ghi: (0, 0)),
                pl.BlockSpec((1, 512), lambda r, glo, ghi: (0, 0)),
                pl.BlockSpec((512, 256), lambda r, glo, ghi: (0, 0)),
                pl.BlockSpec((1, 256), lambda r, glo, ghi: (0, 0)),
                pl.BlockSpec((256, 256), lambda r, glo, ghi: (0, 0)),
                pl.BlockSpec((1, 256), lambda r, glo, ghi: (0, 0)),
                pl.BlockSpec((256, NC), lambda r, glo, ghi: (0, 0)),
                pl.BlockSpec((1, NC), lambda r, glo, ghi: (0, 0)),
            ],
            out_specs=pl.BlockSpec((NUM_GRAPHS, NC),
                                   lambda r, glo, ghi: (0, 0)),
            scratch_shapes=[pltpu.VMEM((NUM_GRAPHS, 512), jnp.float32)],
        ),
        out_shape=jax.ShapeDtypeStruct((NUM_GRAPHS, NC), jnp.float32),
    )(glo, ghi, x2, bT, W0, b0.reshape(1, 512), Wl1, bl1.reshape(1, 256),
      Wl2, bl2.reshape(1, 256), Wl3, bl3.reshape(1, NC))


def kernel(pos, batch, W1a, b1a, W1b, b1b, W1c, b1c, W2a, b2a, W2b, b2b,
           W2c, b2c, W0, b0, Wl1, bl1, Wl2, bl2, Wl3, bl3):
    nbr1, C1, Q1 = _knn_prep(pos, batch, W1a, b1a)
    Q1p = jnp.pad(Q1, ((0, 0), (0, 64)))
    G1 = _sc_gather(Q1p, nbr1)
    x1 = _edge_conv(C1, G1, W1b, b1b, W1c, b1c)

    nbr2, C2, Q2 = _knn_prep(x1, batch, W2a, b2a)
    G2 = _sc_gather(Q2, nbr2)
    x2 = _edge_conv(C2, G2, W2b, b2b, W2c, b2c)

    return _head(x2, batch, W0, b0, Wl1, bl1, Wl2, bl2, Wl3, bl3)
